# baseline (speedup 1.0000x reference)
import functools

import numpy as np
import jax
import jax.numpy as jnp
from jax.experimental import pallas as pl

_PARAM_RAW_DIM = (7, 9, 11, 14, 17)
_MAX_RAW = 17
_PARAM_DIM = 32
_N_TYPES = 5
_EMB_DIM = 16
_LEVELS = (8, 5, 5, 5)

_BBLK = 2048
_IDX_OFFSET = 500

_DNT = (((1,), (1,)), ((), ()))


def _dott(x, w):
    return jax.lax.dot_general(x, w, _DNT, preferred_element_type=jnp.float32)


def _fused(st_ref, p_ref,
           wpe0_ref, bpe0_ref, wpe1_ref, bpe1_ref, wpe2_ref, bpe2_ref,
           wpe3_ref, bpe3_ref, wpe4_ref, bpe4_ref,
           temb_ref, w1_ref, b1_ref, w2_ref, b2_ref, w3_ref, b3_ref,
           w4_ref, b4_ref, wfi_ref, bfi_ref, wfo_ref, bfo_ref,
           wcls_ref, bcls_ref, wcld_ref, bcld_ref,
           wd1_ref, bd1_ref, wd2_ref, bd2_ref, wd3_ref, bd3_ref,
           wr0_ref, br0_ref, wr1_ref, br1_ref, wr2_ref, br2_ref,
           wr3_ref, br3_ref, wr4_ref, br4_ref,
           mstack_ref, shift_ref, hl_ref, off_ref, inv_hw_ref, basis_ref,
           padded_ref, mask_ref, cls_ref, closed_ref, zq_ref, idx_ref):
    f32 = jnp.float32
    dot = functools.partial(jnp.dot, preferred_element_type=f32)

    st = st_ref[...]
    onehot = (st == jax.lax.broadcasted_iota(jnp.int32, (1, _N_TYPES), 1)
              ).astype(f32)

    wpe_refs = (wpe0_ref, wpe1_ref, wpe2_ref, wpe3_ref, wpe4_ref)
    bpe_refs = (bpe0_ref, bpe1_ref, bpe2_ref, bpe3_ref, bpe4_ref)
    wpe = jnp.concatenate(
        [jnp.pad(r[...], ((0, 0), (0, _MAX_RAW - _PARAM_RAW_DIM[t])))
         for t, r in enumerate(wpe_refs)], axis=0)
    bpe = jnp.stack([r[...] for r in bpe_refs], axis=0)

    wr_refs = (wr0_ref, wr1_ref, wr2_ref, wr3_ref, wr4_ref)
    br_refs = (br0_ref, br1_ref, br2_ref, br3_ref, br4_ref)
    wraw = jnp.concatenate(
        [jnp.pad(r[...], ((0, _MAX_RAW - _PARAM_RAW_DIM[t]), (0, 0)))
         for t, r in enumerate(wr_refs)], axis=0)
    braw = jnp.concatenate(
        [jnp.pad(r[...], (0, _MAX_RAW - _PARAM_RAW_DIM[t]))
         for t, r in enumerate(br_refs)], axis=0)

    p = p_ref[...]
    peall = _dott(p, wpe)
    pe = onehot[:, 0:1] * peall[:, 0:_PARAM_DIM]
    for t in range(1, _N_TYPES):
        pe = pe + onehot[:, t:t + 1] * peall[:, t * _PARAM_DIM:(t + 1) * _PARAM_DIM]

    w1 = w1_ref[...]
    temb = temb_ref[...]
    tb1 = (b1_ref[...] + _dott(temb, w1[:, _PARAM_DIM:])
           + _dott(bpe, w1[:, :_PARAM_DIM]))
    h = jnp.maximum(_dott(pe, w1[:, :_PARAM_DIM]) + dot(onehot, tb1), 0.0)
    h = jnp.maximum(_dott(h, w2_ref[...]) + b2_ref[...], 0.0)
    h = jnp.maximum(_dott(h, w3_ref[...]) + b3_ref[...], 0.0)
    z = _dott(h, w4_ref[...]) + b4_ref[...]

    zf = _dott(z, wfi_ref[...]) + bfi_ref[...] + shift_ref[...]
    bounded = jnp.tanh(zf) * hl_ref[...] - off_ref[...]
    rounded = jnp.round(bounded)
    idx = jnp.round(dot(rounded, basis_ref[...]))
    idx_ref[...] = idx.astype(jnp.int32) + _IDX_OFFSET

    codes = rounded * inv_hw_ref[...]
    zq = _dott(codes, wfo_ref[...]) + bfo_ref[...]
    zq_ref[...] = zq

    cls_ref[...] = _dott(zq, wcls_ref[...]) + bcls_ref[...]
    closed_ref[...] = _dott(zq, wcld_ref[...]) + bcld_ref[...]

    wd1 = wd1_ref[...]
    tbd1 = bd1_ref[...] + _dott(temb, wd1[:, 128:])
    hd = jnp.maximum(_dott(zq, wd1[:, :128]) + dot(onehot, tbd1), 0.0)
    hd = jnp.maximum(_dott(hd, wd2_ref[...]) + bd2_ref[...], 0.0)
    pdec = _dott(hd, wd3_ref[...]) + bd3_ref[...]

    outs = _dott(pdec, wraw) + braw
    padded = onehot[:, 0:1] * outs[:, 0:_MAX_RAW]
    for t in range(1, _N_TYPES):
        padded = padded + onehot[:, t:t + 1] * outs[:, t * _MAX_RAW:(t + 1) * _MAX_RAW]
    padded_ref[...] = padded
    mask_ref[...] = dot(onehot, mstack_ref[...]) > 0.5


def kernel(params, surface_type, type_emb, pe_params, enc_params, fsq_in,
           fsq_out, dec_params, cls_params, closed_params, raw_dec_params):
    B = params.shape[0]
    f32 = jnp.float32

    levels = np.array(_LEVELS, dtype=np.float64)
    half_l = ((levels - 1.0) * (1.0 + 1e-3) / 2.0).astype(np.float32)
    offset = np.where(levels % 2 == 0, 0.5, 0.0).astype(np.float32)
    shift = np.arctanh(offset / half_l.astype(np.float64)).astype(np.float32)
    half_width = np.array([l // 2 for l in _LEVELS], dtype=np.float32)
    basis = np.concatenate([[1], np.cumprod(_LEVELS[:-1])]).astype(np.float32)

    (W1, b1), (W2, b2), (W3, b3), (W4, b4) = enc_params
    Wfi, bfi = fsq_in
    Wfo, bfo = fsq_out
    Wc, bc = cls_params
    Wcl, bcl = closed_params
    (Wd1, bd1), (Wd2, bd2), (Wd3, bd3) = dec_params

    mstack = jnp.asarray(
        np.arange(_MAX_RAW)[None, :] < np.array(_PARAM_RAW_DIM)[:, None],
        dtype=f32)
    shift_in = jnp.asarray(shift[None, :])
    hl = jnp.asarray(half_l[None, :])
    off = jnp.asarray(offset[None, :])
    inv_hw = jnp.asarray((1.0 / half_width)[None, :])
    basis_col = jnp.asarray(basis[:, None])

    st2 = surface_type.reshape(B, 1)

    inputs = [st2, params]
    for t in range(_N_TYPES):
        inputs += [pe_params[t][0], pe_params[t][1]]
    inputs += [type_emb, W1, b1, W2, b2, W3, b3, W4, b4,
               Wfi, bfi, Wfo, bfo, Wc, bc, Wcl, bcl,
               Wd1, bd1, Wd2, bd2, Wd3, bd3]
    for t in range(_N_TYPES):
        inputs += [raw_dec_params[t][0], raw_dec_params[t][1]]
    inputs += [mstack, shift_in, hl, off, inv_hw, basis_col]

    row = lambda w: pl.BlockSpec((_BBLK, w), lambda i: (i, 0))
    full = lambda a: pl.BlockSpec(a.shape, lambda i: (0,) * a.ndim)

    out_shapes = (
        jax.ShapeDtypeStruct((B, _MAX_RAW), f32),
        jax.ShapeDtypeStruct((B, _MAX_RAW), jnp.bool_),
        jax.ShapeDtypeStruct((B, _N_TYPES), f32),
        jax.ShapeDtypeStruct((B, 2), f32),
        jax.ShapeDtypeStruct((B, 128), f32),
        jax.ShapeDtypeStruct((B, 1), jnp.int32),
    )
    out_specs = (row(_MAX_RAW), row(_MAX_RAW), row(_N_TYPES), row(2),
                 row(128), row(1))

    outs = pl.pallas_call(
        _fused,
        grid=(B // _BBLK,),
        in_specs=[row(1), row(_MAX_RAW)] + [full(a) for a in inputs[2:]],
        out_specs=out_specs,
        out_shape=out_shapes,
    )(*inputs)

    padded, mask, cls, closed, zq, idx = outs
    return (padded, mask, cls, closed, zq, idx.reshape(B))


# speedup vs baseline: 2.5808x; 2.5808x over previous
import functools

import numpy as np
import jax
import jax.numpy as jnp
from jax.experimental import pallas as pl

_PARAM_RAW_DIM = (7, 9, 11, 14, 17)
_MAX_RAW = 17
_PARAM_DIM = 32
_N_TYPES = 5
_EMB_DIM = 16
_LEVELS = (8, 5, 5, 5)

_BBLK = 2048
_IDX_OFFSET = 500

_F32 = jnp.float32


def _dot(x, w):
    return jax.lax.dot_general(x, w, (((1,), (0,)), ((), ())),
                               preferred_element_type=_F32)


def _dotl(x, w):
    return jax.lax.dot_general(x, w, (((0,), (0,)), ((), ())),
                               preferred_element_type=_F32)


def _col(b):
    return b.reshape(b.shape[0], 1)


def _fused(st_ref, pt_ref,
           wpe0_ref, bpe0_ref, wpe1_ref, bpe1_ref, wpe2_ref, bpe2_ref,
           wpe3_ref, bpe3_ref, wpe4_ref, bpe4_ref,
           temb_ref, w1t_ref, b1_ref, w2_ref, b2_ref, w3_ref, b3_ref,
           w4_ref, b4_ref, wfi_ref, bfi_ref, wfot_ref, bfo_ref,
           wcls_ref, bcls_ref, wcld_ref, bcld_ref,
           wd1t_ref, bd1_ref, wd2_ref, bd2_ref, wd3_ref, bd3_ref,
           wr0_ref, br0_ref, wr1_ref, br1_ref, wr2_ref, br2_ref,
           wr3_ref, br3_ref, wr4_ref, br4_ref,
           mstackt_ref, shift_ref, hl_ref, off_ref, inv_hw_ref, basis_ref,
           paddedt_ref, maskt_ref, clst_ref, closedt_ref, zq_ref, idx_ref):
    st = st_ref[...]
    onehot = (st[None, :] == jax.lax.broadcasted_iota(jnp.int32,
                                                      (_N_TYPES, 1), 0)
              ).astype(_F32)

    wpe_refs = (wpe0_ref, wpe1_ref, wpe2_ref, wpe3_ref, wpe4_ref)
    bpe_refs = (bpe0_ref, bpe1_ref, bpe2_ref, bpe3_ref, bpe4_ref)
    wpet = jnp.concatenate(
        [jnp.pad(r[...], ((0, _MAX_RAW - _PARAM_RAW_DIM[t]), (0, 0)))
         for t, r in enumerate(wpe_refs)], axis=1)
    bpet = jnp.concatenate([_col(r[...]) for r in bpe_refs], axis=1)

    wr_refs = (wr0_ref, wr1_ref, wr2_ref, wr3_ref, wr4_ref)
    br_refs = (br0_ref, br1_ref, br2_ref, br3_ref, br4_ref)
    wraw = jnp.concatenate(
        [jnp.pad(r[...], ((0, _MAX_RAW - _PARAM_RAW_DIM[t]), (0, 0)))
         for t, r in enumerate(wr_refs)], axis=0)
    brawt = jnp.concatenate(
        [jnp.pad(_col(r[...]), ((0, _MAX_RAW - _PARAM_RAW_DIM[t]), (0, 0)))
         for t, r in enumerate(br_refs)], axis=0)

    pt = pt_ref[...]
    peallt = _dotl(wpet, pt)
    pet = onehot[0:1, :] * peallt[0:_PARAM_DIM, :]
    for t in range(1, _N_TYPES):
        pet = pet + onehot[t:t + 1, :] * peallt[t * _PARAM_DIM:(t + 1) * _PARAM_DIM, :]

    w1t = w1t_ref[...]
    tembt = temb_ref[...].T
    tb1t = (_col(b1_ref[...]) + _dotl(w1t[_PARAM_DIM:, :], tembt)
            + _dotl(w1t[:_PARAM_DIM, :], bpet))
    h = jnp.maximum(_dotl(w1t[:_PARAM_DIM, :], pet) + _dot(tb1t, onehot), 0.0)
    h = jnp.maximum(_dot(w2_ref[...], h) + _col(b2_ref[...]), 0.0)
    h = jnp.maximum(_dot(w3_ref[...], h) + _col(b3_ref[...]), 0.0)
    z = _dot(w4_ref[...], h) + _col(b4_ref[...])

    zf = _dot(wfi_ref[...], z) + _col(bfi_ref[...]) + shift_ref[...]
    bounded = jnp.tanh(zf) * hl_ref[...] - off_ref[...]
    rounded = jnp.round(bounded)
    idx = jnp.round(_dot(basis_ref[...], rounded))
    idx_ref[...] = idx.astype(jnp.int32).reshape(-1) + _IDX_OFFSET

    codes = rounded * inv_hw_ref[...]
    wfot = wfot_ref[...]
    bfo = bfo_ref[...]
    zqt = _dotl(wfot, codes) + _col(bfo)
    zq_ref[...] = _dotl(codes, wfot) + bfo[None, :]

    clst_ref[...] = _dot(wcls_ref[...], zqt) + _col(bcls_ref[...])
    closedt_ref[...] = _dot(wcld_ref[...], zqt) + _col(bcld_ref[...])

    wd1t = wd1t_ref[...]
    tbd1t = _col(bd1_ref[...]) + _dotl(wd1t[128:, :], tembt)
    hd = jnp.maximum(_dotl(wd1t[:128, :], zqt) + _dot(tbd1t, onehot), 0.0)
    hd = jnp.maximum(_dot(wd2_ref[...], hd) + _col(bd2_ref[...]), 0.0)
    pdec = _dot(wd3_ref[...], hd) + _col(bd3_ref[...])

    outs = _dot(wraw, pdec) + brawt
    padded = onehot[0:1, :] * outs[0:_MAX_RAW, :]
    for t in range(1, _N_TYPES):
        padded = padded + onehot[t:t + 1, :] * outs[t * _MAX_RAW:(t + 1) * _MAX_RAW, :]
    paddedt_ref[...] = padded
    maskt_ref[...] = _dot(mstackt_ref[...], onehot) > 0.5


def kernel(params, surface_type, type_emb, pe_params, enc_params, fsq_in,
           fsq_out, dec_params, cls_params, closed_params, raw_dec_params):
    B = params.shape[0]

    levels = np.array(_LEVELS, dtype=np.float64)
    half_l = ((levels - 1.0) * (1.0 + 1e-3) / 2.0).astype(np.float32)
    offset = np.where(levels % 2 == 0, 0.5, 0.0).astype(np.float32)
    shift = np.arctanh(offset / half_l.astype(np.float64)).astype(np.float32)
    half_width = np.array([l // 2 for l in _LEVELS], dtype=np.float32)
    basis = np.concatenate([[1], np.cumprod(_LEVELS[:-1])]).astype(np.float32)

    (W1, b1), (W2, b2), (W3, b3), (W4, b4) = enc_params
    Wfi, bfi = fsq_in
    Wfo, bfo = fsq_out
    Wc, bc = cls_params
    Wcl, bcl = closed_params
    (Wd1, bd1), (Wd2, bd2), (Wd3, bd3) = dec_params

    mstackt = jnp.asarray(
        (np.arange(_MAX_RAW)[:, None] < np.array(_PARAM_RAW_DIM)[None, :]),
        dtype=_F32)
    shift_in = jnp.asarray(shift[:, None])
    hl = jnp.asarray(half_l[:, None])
    off = jnp.asarray(offset[:, None])
    inv_hw = jnp.asarray((1.0 / half_width)[:, None])
    basis_row = jnp.asarray(basis[None, :])

    inputs = [surface_type, params.T]
    for t in range(_N_TYPES):
        inputs += [pe_params[t][0].T, pe_params[t][1]]
    inputs += [type_emb, W1.T, b1, W2, b2, W3, b3, W4, b4,
               Wfi, bfi, Wfo.T, bfo, Wc, bc, Wcl, bcl,
               Wd1.T, bd1, Wd2, bd2, Wd3, bd3]
    for t in range(_N_TYPES):
        inputs += [raw_dec_params[t][0], raw_dec_params[t][1]]
    inputs += [mstackt, shift_in, hl, off, inv_hw, basis_row]

    colblk = lambda h: pl.BlockSpec((h, _BBLK), lambda i: (0, i))
    full = lambda a: pl.BlockSpec(a.shape, lambda i: (0,) * a.ndim)

    out_shapes = (
        jax.ShapeDtypeStruct((_MAX_RAW, B), _F32),
        jax.ShapeDtypeStruct((_MAX_RAW, B), jnp.bool_),
        jax.ShapeDtypeStruct((_N_TYPES, B), _F32),
        jax.ShapeDtypeStruct((2, B), _F32),
        jax.ShapeDtypeStruct((B, 128), _F32),
        jax.ShapeDtypeStruct((B,), jnp.int32),
    )
    out_specs = (colblk(_MAX_RAW), colblk(_MAX_RAW), colblk(_N_TYPES),
                 colblk(2), pl.BlockSpec((_BBLK, 128), lambda i: (i, 0)),
                 pl.BlockSpec((_BBLK,), lambda i: (i,)))

    outs = pl.pallas_call(
        _fused,
        grid=(B // _BBLK,),
        in_specs=[pl.BlockSpec((_BBLK,), lambda i: (i,)),
                  colblk(_MAX_RAW)] + [full(a) for a in inputs[2:]],
        out_specs=out_specs,
        out_shape=out_shapes,
    )(*inputs)

    paddedt, maskt, clst, closedt, zq, idx = outs
    return (paddedt.T, maskt.T, clst.T, closedt.T, zq, idx)


# speedup vs baseline: 2.8770x; 1.1148x over previous
import functools

import numpy as np
import jax
import jax.numpy as jnp
from jax.experimental import pallas as pl

_PARAM_RAW_DIM = (7, 9, 11, 14, 17)
_MAX_RAW = 17
_PARAM_DIM = 32
_N_TYPES = 5
_EMB_DIM = 16
_LEVELS = (8, 5, 5, 5)

_BBLK = 4096
_IDX_OFFSET = 500

_F32 = jnp.float32


def _dot(x, w):
    return jax.lax.dot_general(x, w, (((1,), (0,)), ((), ())),
                               preferred_element_type=_F32)


def _dotl(x, w):
    return jax.lax.dot_general(x, w, (((0,), (0,)), ((), ())),
                               preferred_element_type=_F32)


def _col(b):
    return b.reshape(b.shape[0], 1)


def _fused(st_ref, pt_ref,
           wpe0_ref, bpe0_ref, wpe1_ref, bpe1_ref, wpe2_ref, bpe2_ref,
           wpe3_ref, bpe3_ref, wpe4_ref, bpe4_ref,
           temb_ref, w1t_ref, b1_ref, w2_ref, b2_ref, w3_ref, b3_ref,
           w4_ref, b4_ref, wfi_ref, bfi_ref, wfot_ref, bfo_ref,
           wcls_ref, bcls_ref, wcld_ref, bcld_ref,
           wd1t_ref, bd1_ref, wd2_ref, bd2_ref, wd3_ref, bd3_ref,
           wr0_ref, br0_ref, wr1_ref, br1_ref, wr2_ref, br2_ref,
           wr3_ref, br3_ref, wr4_ref, br4_ref,
           mstackt_ref, shift_ref, hl_ref, off_ref, inv_hw_ref, basis_ref,
           paddedt_ref, maskt_ref, clst_ref, closedt_ref, zq_ref, idx_ref):
    st = st_ref[...]
    onehot = (st[None, :] == jax.lax.broadcasted_iota(jnp.int32,
                                                      (_N_TYPES, 1), 0)
              ).astype(_F32)

    wpe_refs = (wpe0_ref, wpe1_ref, wpe2_ref, wpe3_ref, wpe4_ref)
    bpe_refs = (bpe0_ref, bpe1_ref, bpe2_ref, bpe3_ref, bpe4_ref)
    wpet = jnp.concatenate(
        [jnp.pad(r[...], ((0, _MAX_RAW - _PARAM_RAW_DIM[t]), (0, 0)))
         for t, r in enumerate(wpe_refs)], axis=1)
    bpet = jnp.concatenate([_col(r[...]) for r in bpe_refs], axis=1)

    wr_refs = (wr0_ref, wr1_ref, wr2_ref, wr3_ref, wr4_ref)
    br_refs = (br0_ref, br1_ref, br2_ref, br3_ref, br4_ref)
    wraw = jnp.concatenate(
        [jnp.pad(r[...], ((0, _MAX_RAW - _PARAM_RAW_DIM[t]), (0, 0)))
         for t, r in enumerate(wr_refs)], axis=0)
    brawt = jnp.concatenate(
        [jnp.pad(_col(r[...]), ((0, _MAX_RAW - _PARAM_RAW_DIM[t]), (0, 0)))
         for t, r in enumerate(br_refs)], axis=0)

    pt = pt_ref[...]
    peallt = _dotl(wpet, pt)
    pet = onehot[0:1, :] * peallt[0:_PARAM_DIM, :]
    for t in range(1, _N_TYPES):
        pet = pet + onehot[t:t + 1, :] * peallt[t * _PARAM_DIM:(t + 1) * _PARAM_DIM, :]

    w1t = w1t_ref[...]
    tembt = temb_ref[...].T
    tb1t = (_col(b1_ref[...]) + _dotl(w1t[_PARAM_DIM:, :], tembt)
            + _dotl(w1t[:_PARAM_DIM, :], bpet))
    h = jnp.maximum(_dotl(w1t[:_PARAM_DIM, :], pet) + _dot(tb1t, onehot), 0.0)
    h = jnp.maximum(_dot(w2_ref[...], h) + _col(b2_ref[...]), 0.0)
    h = jnp.maximum(_dot(w3_ref[...], h) + _col(b3_ref[...]), 0.0)
    z = _dot(w4_ref[...], h) + _col(b4_ref[...])

    zf = _dot(wfi_ref[...], z) + _col(bfi_ref[...]) + shift_ref[...]
    bounded = jnp.tanh(zf) * hl_ref[...] - off_ref[...]
    rounded = jnp.round(bounded)
    idx = jnp.round(_dot(basis_ref[...], rounded))
    idx_ref[...] = idx.astype(jnp.int32).reshape(-1) + _IDX_OFFSET

    codes = rounded * inv_hw_ref[...]
    wfot = wfot_ref[...]
    bfo = bfo_ref[...]
    zqt = _dotl(wfot, codes) + _col(bfo)
    zq_ref[...] = _dotl(codes, wfot) + bfo[None, :]

    clst_ref[...] = _dot(wcls_ref[...], zqt) + _col(bcls_ref[...])
    closedt_ref[...] = _dot(wcld_ref[...], zqt) + _col(bcld_ref[...])

    wd1t = wd1t_ref[...]
    tbd1t = _col(bd1_ref[...]) + _dotl(wd1t[128:, :], tembt)
    hd = jnp.maximum(_dotl(wd1t[:128, :], zqt) + _dot(tbd1t, onehot), 0.0)
    hd = jnp.maximum(_dot(wd2_ref[...], hd) + _col(bd2_ref[...]), 0.0)
    pdec = _dot(wd3_ref[...], hd) + _col(bd3_ref[...])

    outs = _dot(wraw, pdec) + brawt
    padded = onehot[0:1, :] * outs[0:_MAX_RAW, :]
    for t in range(1, _N_TYPES):
        padded = padded + onehot[t:t + 1, :] * outs[t * _MAX_RAW:(t + 1) * _MAX_RAW, :]
    paddedt_ref[...] = padded
    maskt_ref[...] = _dot(mstackt_ref[...], onehot) > 0.5


def kernel(params, surface_type, type_emb, pe_params, enc_params, fsq_in,
           fsq_out, dec_params, cls_params, closed_params, raw_dec_params):
    B = params.shape[0]

    levels = np.array(_LEVELS, dtype=np.float64)
    half_l = ((levels - 1.0) * (1.0 + 1e-3) / 2.0).astype(np.float32)
    offset = np.where(levels % 2 == 0, 0.5, 0.0).astype(np.float32)
    shift = np.arctanh(offset / half_l.astype(np.float64)).astype(np.float32)
    half_width = np.array([l // 2 for l in _LEVELS], dtype=np.float32)
    basis = np.concatenate([[1], np.cumprod(_LEVELS[:-1])]).astype(np.float32)

    (W1, b1), (W2, b2), (W3, b3), (W4, b4) = enc_params
    Wfi, bfi = fsq_in
    Wfo, bfo = fsq_out
    Wc, bc = cls_params
    Wcl, bcl = closed_params
    (Wd1, bd1), (Wd2, bd2), (Wd3, bd3) = dec_params

    mstackt = jnp.asarray(
        (np.arange(_MAX_RAW)[:, None] < np.array(_PARAM_RAW_DIM)[None, :]),
        dtype=_F32)
    shift_in = jnp.asarray(shift[:, None])
    hl = jnp.asarray(half_l[:, None])
    off = jnp.asarray(offset[:, None])
    inv_hw = jnp.asarray((1.0 / half_width)[:, None])
    basis_row = jnp.asarray(basis[None, :])

    inputs = [surface_type, params.T]
    for t in range(_N_TYPES):
        inputs += [pe_params[t][0].T, pe_params[t][1]]
    inputs += [type_emb, W1.T, b1, W2, b2, W3, b3, W4, b4,
               Wfi, bfi, Wfo.T, bfo, Wc, bc, Wcl, bcl,
               Wd1.T, bd1, Wd2, bd2, Wd3, bd3]
    for t in range(_N_TYPES):
        inputs += [raw_dec_params[t][0], raw_dec_params[t][1]]
    inputs += [mstackt, shift_in, hl, off, inv_hw, basis_row]

    colblk = lambda h: pl.BlockSpec((h, _BBLK), lambda i: (0, i))
    full = lambda a: pl.BlockSpec(a.shape, lambda i: (0,) * a.ndim)

    out_shapes = (
        jax.ShapeDtypeStruct((_MAX_RAW, B), _F32),
        jax.ShapeDtypeStruct((_MAX_RAW, B), jnp.bool_),
        jax.ShapeDtypeStruct((_N_TYPES, B), _F32),
        jax.ShapeDtypeStruct((2, B), _F32),
        jax.ShapeDtypeStruct((B, 128), _F32),
        jax.ShapeDtypeStruct((B,), jnp.int32),
    )
    out_specs = (colblk(_MAX_RAW), colblk(_MAX_RAW), colblk(_N_TYPES),
                 colblk(2), pl.BlockSpec((_BBLK, 128), lambda i: (i, 0)),
                 pl.BlockSpec((_BBLK,), lambda i: (i,)))

    outs = pl.pallas_call(
        _fused,
        grid=(B // _BBLK,),
        in_specs=[pl.BlockSpec((_BBLK,), lambda i: (i,)),
                  colblk(_MAX_RAW)] + [full(a) for a in inputs[2:]],
        out_specs=out_specs,
        out_shape=out_shapes,
    )(*inputs)

    paddedt, maskt, clst, closedt, zq, idx = outs
    return (paddedt.T, maskt.T, clst.T, closedt.T, zq, idx)


# speedup vs baseline: 2.9905x; 1.0395x over previous
import functools

import numpy as np
import jax
import jax.numpy as jnp
from jax.experimental import pallas as pl

_PARAM_RAW_DIM = (7, 9, 11, 14, 17)
_MAX_RAW = 17
_PARAM_DIM = 32
_N_TYPES = 5
_EMB_DIM = 16
_LEVELS = (8, 5, 5, 5)

_BBLK = 8192
_IDX_OFFSET = 500

_F32 = jnp.float32


def _dot(x, w):
    return jax.lax.dot_general(x, w, (((1,), (0,)), ((), ())),
                               preferred_element_type=_F32)


def _dotl(x, w):
    return jax.lax.dot_general(x, w, (((0,), (0,)), ((), ())),
                               preferred_element_type=_F32)


def _col(b):
    return b.reshape(b.shape[0], 1)


def _fused(st_ref, pt_ref,
           wpe0_ref, bpe0_ref, wpe1_ref, bpe1_ref, wpe2_ref, bpe2_ref,
           wpe3_ref, bpe3_ref, wpe4_ref, bpe4_ref,
           temb_ref, w1t_ref, b1_ref, w2_ref, b2_ref, w3_ref, b3_ref,
           w4_ref, b4_ref, wfi_ref, bfi_ref, wfot_ref, bfo_ref,
           wcls_ref, bcls_ref, wcld_ref, bcld_ref,
           wd1t_ref, bd1_ref, wd2_ref, bd2_ref, wd3_ref, bd3_ref,
           wr0_ref, br0_ref, wr1_ref, br1_ref, wr2_ref, br2_ref,
           wr3_ref, br3_ref, wr4_ref, br4_ref,
           mstackt_ref, shift_ref, hl_ref, off_ref, inv_hw_ref, basis_ref,
           paddedt_ref, maskt_ref, clst_ref, closedt_ref, zq_ref, idx_ref):
    st = st_ref[...]
    onehot = (st[None, :] == jax.lax.broadcasted_iota(jnp.int32,
                                                      (_N_TYPES, 1), 0)
              ).astype(_F32)

    wpe_refs = (wpe0_ref, wpe1_ref, wpe2_ref, wpe3_ref, wpe4_ref)
    bpe_refs = (bpe0_ref, bpe1_ref, bpe2_ref, bpe3_ref, bpe4_ref)
    wpet = jnp.concatenate(
        [jnp.pad(r[...], ((0, _MAX_RAW - _PARAM_RAW_DIM[t]), (0, 0)))
         for t, r in enumerate(wpe_refs)], axis=1)
    bpet = jnp.concatenate([_col(r[...]) for r in bpe_refs], axis=1)

    wr_refs = (wr0_ref, wr1_ref, wr2_ref, wr3_ref, wr4_ref)
    br_refs = (br0_ref, br1_ref, br2_ref, br3_ref, br4_ref)
    wraw = jnp.concatenate(
        [jnp.pad(r[...], ((0, _MAX_RAW - _PARAM_RAW_DIM[t]), (0, 0)))
         for t, r in enumerate(wr_refs)], axis=0)
    brawt = jnp.concatenate(
        [jnp.pad(_col(r[...]), ((0, _MAX_RAW - _PARAM_RAW_DIM[t]), (0, 0)))
         for t, r in enumerate(br_refs)], axis=0)

    pt = pt_ref[...]
    peallt = _dotl(wpet, pt)
    pet = onehot[0:1, :] * peallt[0:_PARAM_DIM, :]
    for t in range(1, _N_TYPES):
        pet = pet + onehot[t:t + 1, :] * peallt[t * _PARAM_DIM:(t + 1) * _PARAM_DIM, :]

    w1t = w1t_ref[...]
    tembt = temb_ref[...].T
    tb1t = (_col(b1_ref[...]) + _dotl(w1t[_PARAM_DIM:, :], tembt)
            + _dotl(w1t[:_PARAM_DIM, :], bpet))
    h = jnp.maximum(_dotl(w1t[:_PARAM_DIM, :], pet) + _dot(tb1t, onehot), 0.0)
    h = jnp.maximum(_dot(w2_ref[...], h) + _col(b2_ref[...]), 0.0)
    h = jnp.maximum(_dot(w3_ref[...], h) + _col(b3_ref[...]), 0.0)
    z = _dot(w4_ref[...], h) + _col(b4_ref[...])

    zf = _dot(wfi_ref[...], z) + _col(bfi_ref[...]) + shift_ref[...]
    bounded = jnp.tanh(zf) * hl_ref[...] - off_ref[...]
    rounded = jnp.round(bounded)
    idx = jnp.round(_dot(basis_ref[...], rounded))
    idx_ref[...] = idx.astype(jnp.int32).reshape(-1) + _IDX_OFFSET

    codes = rounded * inv_hw_ref[...]
    wfot = wfot_ref[...]
    bfo = bfo_ref[...]
    zqt = _dotl(wfot, codes) + _col(bfo)
    zq_ref[...] = _dotl(codes, wfot) + bfo[None, :]

    clst_ref[...] = _dot(wcls_ref[...], zqt) + _col(bcls_ref[...])
    closedt_ref[...] = _dot(wcld_ref[...], zqt) + _col(bcld_ref[...])

    wd1t = wd1t_ref[...]
    tbd1t = _col(bd1_ref[...]) + _dotl(wd1t[128:, :], tembt)
    hd = jnp.maximum(_dotl(wd1t[:128, :], zqt) + _dot(tbd1t, onehot), 0.0)
    hd = jnp.maximum(_dot(wd2_ref[...], hd) + _col(bd2_ref[...]), 0.0)
    pdec = _dot(wd3_ref[...], hd) + _col(bd3_ref[...])

    outs = _dot(wraw, pdec) + brawt
    padded = onehot[0:1, :] * outs[0:_MAX_RAW, :]
    for t in range(1, _N_TYPES):
        padded = padded + onehot[t:t + 1, :] * outs[t * _MAX_RAW:(t + 1) * _MAX_RAW, :]
    paddedt_ref[...] = padded
    maskt_ref[...] = _dot(mstackt_ref[...], onehot) > 0.5


def kernel(params, surface_type, type_emb, pe_params, enc_params, fsq_in,
           fsq_out, dec_params, cls_params, closed_params, raw_dec_params):
    B = params.shape[0]

    levels = np.array(_LEVELS, dtype=np.float64)
    half_l = ((levels - 1.0) * (1.0 + 1e-3) / 2.0).astype(np.float32)
    offset = np.where(levels % 2 == 0, 0.5, 0.0).astype(np.float32)
    shift = np.arctanh(offset / half_l.astype(np.float64)).astype(np.float32)
    half_width = np.array([l // 2 for l in _LEVELS], dtype=np.float32)
    basis = np.concatenate([[1], np.cumprod(_LEVELS[:-1])]).astype(np.float32)

    (W1, b1), (W2, b2), (W3, b3), (W4, b4) = enc_params
    Wfi, bfi = fsq_in
    Wfo, bfo = fsq_out
    Wc, bc = cls_params
    Wcl, bcl = closed_params
    (Wd1, bd1), (Wd2, bd2), (Wd3, bd3) = dec_params

    mstackt = jnp.asarray(
        (np.arange(_MAX_RAW)[:, None] < np.array(_PARAM_RAW_DIM)[None, :]),
        dtype=_F32)
    shift_in = jnp.asarray(shift[:, None])
    hl = jnp.asarray(half_l[:, None])
    off = jnp.asarray(offset[:, None])
    inv_hw = jnp.asarray((1.0 / half_width)[:, None])
    basis_row = jnp.asarray(basis[None, :])

    inputs = [surface_type, params.T]
    for t in range(_N_TYPES):
        inputs += [pe_params[t][0].T, pe_params[t][1]]
    inputs += [type_emb, W1.T, b1, W2, b2, W3, b3, W4, b4,
               Wfi, bfi, Wfo.T, bfo, Wc, bc, Wcl, bcl,
               Wd1.T, bd1, Wd2, bd2, Wd3, bd3]
    for t in range(_N_TYPES):
        inputs += [raw_dec_params[t][0], raw_dec_params[t][1]]
    inputs += [mstackt, shift_in, hl, off, inv_hw, basis_row]

    colblk = lambda h: pl.BlockSpec((h, _BBLK), lambda i: (0, i))
    full = lambda a: pl.BlockSpec(a.shape, lambda i: (0,) * a.ndim)

    out_shapes = (
        jax.ShapeDtypeStruct((_MAX_RAW, B), _F32),
        jax.ShapeDtypeStruct((_MAX_RAW, B), jnp.bool_),
        jax.ShapeDtypeStruct((_N_TYPES, B), _F32),
        jax.ShapeDtypeStruct((2, B), _F32),
        jax.ShapeDtypeStruct((B, 128), _F32),
        jax.ShapeDtypeStruct((B,), jnp.int32),
    )
    out_specs = (colblk(_MAX_RAW), colblk(_MAX_RAW), colblk(_N_TYPES),
                 colblk(2), pl.BlockSpec((_BBLK, 128), lambda i: (i, 0)),
                 pl.BlockSpec((_BBLK,), lambda i: (i,)))

    outs = pl.pallas_call(
        _fused,
        grid=(B // _BBLK,),
        in_specs=[pl.BlockSpec((_BBLK,), lambda i: (i,)),
                  colblk(_MAX_RAW)] + [full(a) for a in inputs[2:]],
        out_specs=out_specs,
        out_shape=out_shapes,
    )(*inputs)

    paddedt, maskt, clst, closedt, zq, idx = outs
    return (paddedt.T, maskt.T, clst.T, closedt.T, zq, idx)
